# Initial kernel scaffold; baseline (speedup 1.0000x reference)
import jax
import jax.numpy as jnp
from jax.experimental import pallas as pl


def kernel(x, edge_index, W1l, W1r, att1, b1, gamma, beta, W2l, W2r, att2, b2, W3, b3):
    raise NotImplementedError("write your pallas kernel here")


# speedup vs baseline: 30.9511x; 30.9511x over previous
import functools

import jax
import jax.numpy as jnp
from jax import lax
from jax.experimental import pallas as pl
from jax.experimental.pallas import tpu as pltpu
from jax.experimental.pallas import tpu_sc as plsc

f32 = jnp.float32

NC = 2
NS = 16
NW = NC * NS

@functools.lru_cache(maxsize=None)
def _mesh():
    return plsc.VectorSubcoreMesh(core_axis_name="c", subcore_axis_name="s",
                                  num_cores=NC, num_subcores=NS)


def _shuf_xor(v, k, lane):
    idx = lane ^ k
    return lax.gather(
        v, idx[:, None],
        lax.GatherDimensionNumbers(offset_dims=(), collapsed_slice_dims=(0,),
                                   start_index_map=(0,)),
        (1,), mode=lax.GatherScatterMode.PROMISE_IN_BOUNDS)


def _bcast_max(v, lane):
    for k in (1, 2, 4, 8):
        v = jnp.maximum(v, _shuf_xor(v, k, lane))
    return v


def _tc_project(x, w_cat, n, d):
    def body(x_ref, w_ref, xl_ref, xr_ref):
        y = jnp.dot(x_ref[...], w_ref[...], preferred_element_type=f32)
        xl_ref[...] = y[:, :64]
        xr_ref[...] = y[:, 64:]

    return pl.pallas_call(
        body,
        out_shape=[jax.ShapeDtypeStruct((n, 64), f32),
                   jax.ShapeDtypeStruct((n, 64), f32)],
    )(x, w_cat)


def _tc_mid(part1, b1, gamma, beta, expander, w24, w3, b3, n):
    def body(p_ref, b1_ref, g_ref, be_ref, ex_ref, w24_ref, w3_ref, b3_ref,
             t2_ref, y3_ref):
        p = p_ref[0] + p_ref[1]
        num = p[:, :64]
        den = p[:, 64:68]
        dinv = 1.0 / (den + 1e-16)
        denb = jnp.dot(dinv, ex_ref[...], preferred_element_type=f32)
        pre = num * denb + b1_ref[...]
        h = jnp.where(pre > 0, pre, jnp.exp(pre) - 1.0)
        mean = jnp.mean(h, axis=0)
        cent = h - mean
        var = jnp.mean(cent * cent, axis=0)
        hbn = g_ref[...] * cent / jnp.sqrt(var + 1e-5) + be_ref[...]
        t2_ref[...] = jnp.dot(hbn, w24_ref[...], preferred_element_type=f32)
        y3_ref[...] = (jnp.dot(hbn, w3_ref[...], preferred_element_type=f32)
                       + b3_ref[...])

    return pl.pallas_call(
        body,
        out_shape=[jax.ShapeDtypeStruct((n, 16), f32),
                   jax.ShapeDtypeStruct((n, 2), f32)],
    )(part1, b1, gamma, beta, expander, w24, w3, b3)


def _tc_final(part2, y3, b2, n):
    def body(p_ref, y3_ref, b2_ref, o_ref):
        p = p_ref[0] + p_ref[1]
        num = p[:, 0:2]
        den = p[:, 2:4]
        z = num / (den + 1e-16) + b2_ref[...] + y3_ref[...]
        o_ref[...] = jnp.where(z > 0, z, jnp.exp(z) - 1.0)

    return pl.pallas_call(
        body,
        out_shape=jax.ShapeDtypeStruct((n, 2), f32),
    )(part2, y3, b2)


def _sc_logits1(xl, xr, src, dst, att1, e_total, blk):
    ew = e_total // NW
    nchunk = ew // blk

    @functools.partial(
        pl.kernel,
        mesh=_mesh(),
        compiler_params=pltpu.CompilerParams(needs_layout_passes=False, use_tc_tiling_on_sc=False),
        out_type=[jax.ShapeDtypeStruct((e_total * 4,), f32),
                  jax.ShapeDtypeStruct((NW * 16,), f32)],
        scratch_types=[
            pltpu.VMEM((blk,), jnp.int32),
            pltpu.VMEM((blk,), jnp.int32),
            pltpu.VMEM((blk, 64), f32),
            pltpu.VMEM((blk, 64), f32),
            pltpu.VMEM((blk * 4,), f32),
            pltpu.VMEM((4, 16), f32),
            pltpu.VMEM((16,), f32),
            pltpu.SemaphoreType.DMA,
        ],
    )
    def k(xl_h, xr_h, src_h, dst_h, att_h, lg_h, wmax_h,
          srcv, dstv, xlv, xrv, lgv, attv, wmv, sem):
        c = lax.axis_index("c")
        s = lax.axis_index("s")
        wid = c * NS + s
        pltpu.sync_copy(att_h, attv)
        att_rows = [attv[h, :] for h in range(4)]
        base0 = wid * ew
        lane = lax.iota(jnp.int32, 16)
        czero = jnp.zeros((16,), jnp.int32)

        def chunk(i, vmax):
            base = base0 + i * blk
            pltpu.sync_copy(src_h.at[pl.ds(base, blk)], srcv)
            pltpu.sync_copy(dst_h.at[pl.ds(base, blk)], dstv)
            cp1 = pltpu.async_copy(xl_h.at[srcv], xlv, sem)
            cp2 = pltpu.async_copy(xr_h.at[dstv], xrv, sem)
            cp1.wait()
            cp2.wait()

            def grp(g, vm):
                eidx = lane + g * 16
                for h in range(4):
                    acc = jnp.zeros((16,), f32)
                    for cc in range(16):
                        cs = czero + (h * 16 + cc)
                        z = (plsc.load_gather(xlv, [eidx, cs])
                             + plsc.load_gather(xrv, [eidx, cs]))
                        z = jnp.maximum(z, 0.2 * z)
                        acc = acc + z * att_rows[h][cc]
                    lgv[pl.ds(h * blk + g * 16, 16)] = acc
                    vm = jnp.maximum(vm, acc)
                return vm

            vmax = lax.fori_loop(0, blk // 16, grp, vmax)
            for h in range(4):
                pltpu.sync_copy(lgv.at[pl.ds(h * blk, blk)],
                                lg_h.at[pl.ds(h * e_total + base, blk)])
            return vmax

        vmax = lax.fori_loop(0, nchunk, chunk,
                             jnp.full((16,), -3e38, dtype=f32))
        wmv[...] = vmax
        pltpu.sync_copy(wmv, wmax_h.at[pl.ds(wid * 16, 16)])

    return k(xl, xr, src, dst, att1)


def _sc_accum1(xl, src, dst, lg, wmax, zin, e_total, n, blk):
    ew = e_total // NW
    nchunk = ew // blk
    stripe = n // NS

    @functools.partial(
        pl.kernel,
        mesh=_mesh(),
        compiler_params=pltpu.CompilerParams(needs_layout_passes=False, use_tc_tiling_on_sc=False),
        out_type=[jax.ShapeDtypeStruct((NC, n, 80), f32)],
        scratch_types=[
            pltpu.VMEM((blk,), jnp.int32),
            pltpu.VMEM((blk,), jnp.int32),
            pltpu.VMEM((blk, 64), f32),
            pltpu.VMEM((blk * 4,), f32),
            pltpu.VMEM((blk * 4,), f32),
            pltpu.VMEM((blk, 80), f32),
            pltpu.VMEM((NW * 16,), f32),
            pltpu.VMEM_SHARED((n, 80), f32),
            pltpu.SemaphoreType.DMA,
        ],
    )
    def k(xl_h, src_h, dst_h, lg_h, wmax_h, zin_h, out_h,
          srcv, dstv, xlv, lgcv, exf, accv, wmv, sh, sem):
        c = lax.axis_index("c")
        s = lax.axis_index("s")
        wid = c * NS + s
        lane = lax.iota(jnp.int32, 16)
        lane4blk = jnp.minimum(lane, 3) * blk

        pltpu.sync_copy(wmax_h, wmv)

        def mx(j, vm):
            return jnp.maximum(vm, wmv[pl.ds(j * 16, 16)])

        vm = lax.fori_loop(0, NW, mx, jnp.full((16,), -3e38, dtype=f32))
        gmaxv = _bcast_max(vm, lane)

        pltpu.sync_copy(zin_h.at[pl.ds(s * stripe, stripe)],
                        sh.at[pl.ds(s * stripe, stripe)])
        pltpu.sync_copy(zin_h.at[pl.ds(0, blk)], accv)
        plsc.subcore_barrier()

        base0 = wid * ew

        def chunk(i, _):
            base = base0 + i * blk
            pltpu.sync_copy(src_h.at[pl.ds(base, blk)], srcv)
            pltpu.sync_copy(dst_h.at[pl.ds(base, blk)], dstv)
            for h in range(4):
                pltpu.sync_copy(lg_h.at[pl.ds(h * e_total + base, blk)],
                                lgcv.at[pl.ds(h * blk, blk)])
            cp = pltpu.async_copy(xl_h.at[srcv], xlv, sem)

            def expj(j, _2):
                exf[pl.ds(j * 16, 16)] = jnp.exp(
                    lgcv[pl.ds(j * 16, 16)] - gmaxv)
                return 0

            lax.fori_loop(0, blk * 4 // 16, expj, 0)
            cp.wait()

            def edge(e, _2):
                exvec = plsc.load_gather(exf, [lane4blk + e])
                for h in range(4):
                    accv[e, pl.ds(h * 16, 16)] = (
                        xlv[e, pl.ds(h * 16, 16)] * exvec[h])
                accv[e, pl.ds(64, 16)] = exvec
                return 0

            lax.fori_loop(0, blk, edge, 0)
            pltpu.sync_copy(accv, sh.at[dstv], add=True)
            return 0

        lax.fori_loop(0, nchunk, chunk, 0)
        plsc.subcore_barrier()
        pltpu.sync_copy(sh.at[pl.ds(s * stripe, stripe)],
                        out_h.at[c, pl.ds(s * stripe, stripe)])

    return k(xl, src, dst, lg, wmax, zin)[0]


def _sc_logits2(t2, src, dst, att2, e_total, blk):
    ew = e_total // NW
    nchunk = ew // blk
    ngrp = blk // 16

    @functools.partial(
        pl.kernel,
        mesh=_mesh(),
        compiler_params=pltpu.CompilerParams(needs_layout_passes=False, use_tc_tiling_on_sc=False),
        out_type=[jax.ShapeDtypeStruct((e_total,), f32),
                  jax.ShapeDtypeStruct((NW * 16,), f32)],
        scratch_types=[
            pltpu.VMEM((blk,), jnp.int32),
            pltpu.VMEM((blk,), jnp.int32),
            pltpu.VMEM((blk, 16), f32),
            pltpu.VMEM((blk, 16), f32),
            pltpu.VMEM((blk,), f32),
            pltpu.VMEM((16,), f32),
            pltpu.VMEM((16,), f32),
            pltpu.SemaphoreType.DMA,
        ],
    )
    def k(t2_h, src_h, dst_h, att_h, lg_h, wmax_h,
          srcv, dstv, tsv, tdv, lgv, attv, wmv, sem):
        c = lax.axis_index("c")
        s = lax.axis_index("s")
        wid = c * NS + s
        pltpu.sync_copy(att_h, attv)
        av = attv[...]
        t0 = av[0]
        t1 = av[1]
        iota = lax.iota(jnp.int32, 16)
        c0 = jnp.zeros((16,), jnp.int32)
        c1 = c0 + 1
        c2 = c0 + 2
        c3 = c0 + 3
        base0 = wid * ew

        def chunk(i, vmax):
            base = base0 + i * blk
            pltpu.sync_copy(src_h.at[pl.ds(base, blk)], srcv)
            pltpu.sync_copy(dst_h.at[pl.ds(base, blk)], dstv)
            cp1 = pltpu.async_copy(t2_h.at[srcv], tsv, sem)
            cp2 = pltpu.async_copy(t2_h.at[dstv], tdv, sem)
            cp1.wait()
            cp2.wait()

            def grp(g, vm):
                i0 = iota + g * 16
                a0 = plsc.load_gather(tsv, [i0, c0])
                a1 = plsc.load_gather(tsv, [i0, c1])
                d0 = plsc.load_gather(tdv, [i0, c2])
                d1 = plsc.load_gather(tdv, [i0, c3])
                z0 = a0 + d0
                z0 = jnp.maximum(z0, 0.2 * z0)
                z1 = a1 + d1
                z1 = jnp.maximum(z1, 0.2 * z1)
                lgvec = z0 * t0 + z1 * t1
                lgv[pl.ds(g * 16, 16)] = lgvec
                return jnp.maximum(vm, lgvec)

            vmax = lax.fori_loop(0, ngrp, grp, vmax)
            pltpu.sync_copy(lgv, lg_h.at[pl.ds(base, blk)])
            return vmax

        vmax = lax.fori_loop(0, nchunk, chunk,
                             jnp.full((16,), -3e38, dtype=f32))
        wmv[...] = vmax
        pltpu.sync_copy(wmv, wmax_h.at[pl.ds(wid * 16, 16)])

    return k(t2, src, dst, att2)


def _sc_accum2(t2, src, dst, lg, wmax, zin, e_total, n, blk):
    ew = e_total // NW
    nchunk = ew // blk
    ngrp = blk // 16
    stripe = n // NS

    @functools.partial(
        pl.kernel,
        mesh=_mesh(),
        compiler_params=pltpu.CompilerParams(needs_layout_passes=False, use_tc_tiling_on_sc=False),
        out_type=[jax.ShapeDtypeStruct((NC, n, 8), f32)],
        scratch_types=[
            pltpu.VMEM((blk,), jnp.int32),
            pltpu.VMEM((blk,), jnp.int32),
            pltpu.VMEM((blk, 16), f32),
            pltpu.VMEM((blk,), f32),
            pltpu.VMEM((blk, 8), f32),
            pltpu.VMEM((NW * 16,), f32),
            pltpu.VMEM_SHARED((n, 8), f32),
            pltpu.SemaphoreType.DMA,
        ],
    )
    def k(t2_h, src_h, dst_h, lg_h, wmax_h, zin_h, out_h,
          srcv, dstv, tsv, lgv, accv, wmv, sh, sem):
        c = lax.axis_index("c")
        s = lax.axis_index("s")
        wid = c * NS + s

        pltpu.sync_copy(wmax_h, wmv)

        def mx(j, vm):
            return jnp.maximum(vm, wmv[pl.ds(j * 16, 16)])

        vm = lax.fori_loop(0, NW, mx, jnp.full((16,), -3e38, dtype=f32))
        gmaxv = _bcast_max(vm, lax.iota(jnp.int32, 16))

        pltpu.sync_copy(zin_h.at[pl.ds(s * stripe, stripe)],
                        sh.at[pl.ds(s * stripe, stripe)])
        pltpu.sync_copy(zin_h.at[pl.ds(0, blk)], accv)
        plsc.subcore_barrier()

        iota = lax.iota(jnp.int32, 16)
        c0 = jnp.zeros((16,), jnp.int32)
        c1 = c0 + 1
        c2 = c0 + 2
        c3 = c0 + 3
        base0 = wid * ew

        def chunk(i, _):
            base = base0 + i * blk
            pltpu.sync_copy(src_h.at[pl.ds(base, blk)], srcv)
            pltpu.sync_copy(dst_h.at[pl.ds(base, blk)], dstv)
            pltpu.sync_copy(lg_h.at[pl.ds(base, blk)], lgv)
            cp = pltpu.async_copy(t2_h.at[srcv], tsv, sem)
            cp.wait()

            def grp(g, _2):
                i0 = iota + g * 16
                ex = jnp.exp(lgv[pl.ds(g * 16, 16)] - gmaxv)
                a0 = plsc.load_gather(tsv, [i0, c0])
                a1 = plsc.load_gather(tsv, [i0, c1])
                plsc.store_scatter(accv, [i0, c0], a0 * ex)
                plsc.store_scatter(accv, [i0, c1], a1 * ex)
                plsc.store_scatter(accv, [i0, c2], ex)
                plsc.store_scatter(accv, [i0, c3], ex)
                return 0

            lax.fori_loop(0, ngrp, grp, 0)
            pltpu.sync_copy(accv, sh.at[dstv], add=True)
            return 0

        lax.fori_loop(0, nchunk, chunk, 0)
        plsc.subcore_barrier()
        pltpu.sync_copy(sh.at[pl.ds(s * stripe, stripe)],
                        out_h.at[c, pl.ds(s * stripe, stripe)])

    return k(t2, src, dst, lg, wmax, zin)[0]


USE_SC1 = True
USE_SA1 = True
USE_SC2 = True
USE_SA2 = True
USE_TC = True


def _jnp_logits1(xl, xr, src, dst, att1, e_total):
    xls = xl[src].reshape(e_total, 4, 16)
    xrd = xr[dst].reshape(e_total, 4, 16)
    z = xls + xrd
    z = jnp.maximum(z, 0.2 * z)
    lg = (z * att1[None]).sum(-1)
    wmax = jnp.full((NW * 16,), lg.max(), f32)
    return lg.T.reshape(-1), wmax


def _jnp_accum1(xl, src, dst, lgp, wmax, e_total, n):
    lg = lgp.reshape(4, e_total).T
    ex = jnp.exp(lg - wmax.max())
    xls = xl[src].reshape(e_total, 4, 16)
    num = jax.ops.segment_sum((xls * ex[:, :, None]).reshape(e_total, 64),
                              dst, num_segments=n)
    den = jax.ops.segment_sum(ex, dst, num_segments=n)
    part = jnp.zeros((NC, n, 80), f32)
    part = part.at[0, :, :64].set(num).at[0, :, 64:68].set(den)
    return part


def _jnp_mid(part1, b1, gamma, beta, w24, w3, b3, n):
    p = part1[0] + part1[1]
    num = p[:, :64]
    den = p[:, 64:68]
    denb = jnp.repeat(1.0 / (den + 1e-16), 16, axis=1)
    pre = num * denb + b1
    h = jnp.where(pre > 0, pre, jnp.exp(pre) - 1)
    mean = h.mean(0); cent = h - mean; var = (cent * cent).mean(0)
    hbn = gamma * cent / jnp.sqrt(var + 1e-5) + beta
    return hbn @ w24, hbn @ w3 + b3


def _jnp_logits2(t2, src, dst, att2p, e_total):
    ts = t2[src]; td = t2[dst]
    z0 = ts[:, 0] + td[:, 2]; z0 = jnp.maximum(z0, 0.2 * z0)
    z1 = ts[:, 1] + td[:, 3]; z1 = jnp.maximum(z1, 0.2 * z1)
    lg2 = z0 * att2p[0] + z1 * att2p[1]
    return lg2, jnp.full((NW * 16,), lg2.max(), f32)


def _jnp_accum2(t2, src, dst, lg2, wmax, e_total, n):
    ex2 = jnp.exp(lg2 - wmax.max())
    ts = t2[src]
    num2 = jax.ops.segment_sum(ts[:, :2] * ex2[:, None], dst, num_segments=n)
    den2 = jax.ops.segment_sum(ex2, dst, num_segments=n)
    part = jnp.zeros((NC, n, 8), f32)
    part = part.at[0, :, :2].set(num2)
    part = part.at[0, :, 2].set(den2).at[0, :, 3].set(den2)
    return part


def _jnp_final(part2, y3, b2):
    p = part2[0] + part2[1]
    zf = p[:, 0:2] / (p[:, 2:4] + 1e-16) + b2 + y3
    return jnp.where(zf > 0, zf, jnp.exp(zf) - 1)


def kernel(x, edge_index, W1l, W1r, att1, b1, gamma, beta, W2l, W2r, att2,
           b2, W3, b3):
    n, d = x.shape
    e_total = edge_index.shape[1]
    blk = 400
    assert e_total % (NW * blk) == 0 and n % NS == 0

    src = edge_index[0]
    dst = edge_index[1]
    w1cat = jnp.concatenate([W1l, W1r], axis=1)
    w24 = jnp.concatenate(
        [W2l, W2r, jnp.zeros((64, 12), f32)], axis=1)
    expander = jnp.repeat(jnp.eye(4, dtype=f32), 16, axis=1)
    zin1 = jnp.zeros((n, 80), f32)
    zin2 = jnp.zeros((n, 8), f32)

    att2p = jnp.zeros((16,), f32).at[:2].set(att2.reshape(-1))
    if USE_TC:
        xl, xr = _tc_project(x, w1cat, n, d)
    else:
        y = x @ w1cat
        xl, xr = y[:, :64], y[:, 64:]
    if USE_SC1:
        lg1, wmax1 = _sc_logits1(xl, xr, src, dst, att1, e_total, blk)
    else:
        lg1, wmax1 = _jnp_logits1(xl, xr, src, dst, att1, e_total)
    if USE_SA1:
        part1 = _sc_accum1(xl, src, dst, lg1, wmax1, zin1, e_total, n, blk)
    else:
        part1 = _jnp_accum1(xl, src, dst, lg1, wmax1, e_total, n)
    if USE_TC:
        t2, y3 = _tc_mid(part1, b1, gamma, beta, expander, w24, W3, b3, n)
    else:
        t2, y3 = _jnp_mid(part1, b1, gamma, beta, w24, W3, b3, n)
    if USE_SC2:
        lg2, wmax2 = _sc_logits2(t2, src, dst, att2p, e_total, blk)
    else:
        lg2, wmax2 = _jnp_logits2(t2, src, dst, att2p, e_total)
    if USE_SA2:
        part2 = _sc_accum2(t2, src, dst, lg2, wmax2, zin2, e_total, n, blk)
    else:
        part2 = _jnp_accum2(t2, src, dst, lg2, wmax2, e_total, n)
    if USE_TC:
        return _tc_final(part2, y3, b2, n)
    return _jnp_final(part2, y3, b2)


# speedup vs baseline: 58.7039x; 1.8967x over previous
import functools

import jax
import jax.numpy as jnp
from jax import lax
from jax.experimental import pallas as pl
from jax.experimental.pallas import tpu as pltpu
from jax.experimental.pallas import tpu_sc as plsc

f32 = jnp.float32

NC = 2
NS = 16
NW = NC * NS

@functools.lru_cache(maxsize=None)
def _mesh():
    return plsc.VectorSubcoreMesh(core_axis_name="c", subcore_axis_name="s",
                                  num_cores=NC, num_subcores=NS)


def _shuf_idx(v, idx):
    return lax.gather(
        v, idx[:, None],
        lax.GatherDimensionNumbers(offset_dims=(), collapsed_slice_dims=(0,),
                                   start_index_map=(0,)),
        (1,), mode=lax.GatherScatterMode.PROMISE_IN_BOUNDS)


def _shuf_xor(v, k, lane):
    idx = lane ^ k
    return lax.gather(
        v, idx[:, None],
        lax.GatherDimensionNumbers(offset_dims=(), collapsed_slice_dims=(0,),
                                   start_index_map=(0,)),
        (1,), mode=lax.GatherScatterMode.PROMISE_IN_BOUNDS)


def _bcast_max(v, lane):
    for k in (1, 2, 4, 8):
        v = jnp.maximum(v, _shuf_xor(v, k, lane))
    return v


def _tc_project(x, w_cat, n, d):
    def body(x_ref, w_ref, xl_ref, xr_ref):
        y = jnp.dot(x_ref[...], w_ref[...], preferred_element_type=f32)
        xl_ref[...] = y[:, :64]
        xr_ref[...] = y[:, 64:]

    return pl.pallas_call(
        body,
        out_shape=[jax.ShapeDtypeStruct((n, 64), f32),
                   jax.ShapeDtypeStruct((n, 64), f32)],
    )(x, w_cat)


def _tc_mid(part1, b1, gamma, beta, expander, w24, w3, b3, n):
    def body(p_ref, b1_ref, g_ref, be_ref, ex_ref, w24_ref, w3_ref, b3_ref,
             t2_ref, y3_ref):
        p = p_ref[0] + p_ref[1]
        num = p[:, :64]
        den = p[:, 64:68]
        dinv = 1.0 / (den + 1e-16)
        denb = jnp.dot(dinv, ex_ref[...], preferred_element_type=f32)
        pre = num * denb + b1_ref[...]
        h = jnp.where(pre > 0, pre, jnp.exp(pre) - 1.0)
        mean = jnp.mean(h, axis=0)
        cent = h - mean
        var = jnp.mean(cent * cent, axis=0)
        hbn = g_ref[...] * cent / jnp.sqrt(var + 1e-5) + be_ref[...]
        t2_ref[...] = jnp.dot(hbn, w24_ref[...], preferred_element_type=f32)
        y3_ref[...] = (jnp.dot(hbn, w3_ref[...], preferred_element_type=f32)
                       + b3_ref[...])

    return pl.pallas_call(
        body,
        out_shape=[jax.ShapeDtypeStruct((n, 16), f32),
                   jax.ShapeDtypeStruct((n, 2), f32)],
    )(part1, b1, gamma, beta, expander, w24, w3, b3)


def _tc_final(part2, y3, b2, n):
    def body(p_ref, y3_ref, b2_ref, o_ref):
        p = p_ref[0] + p_ref[1]
        num = p[:, 0:2]
        den = p[:, 2:4]
        z = num / (den + 1e-16) + b2_ref[...] + y3_ref[...]
        o_ref[...] = jnp.where(z > 0, z, jnp.exp(z) - 1.0)

    return pl.pallas_call(
        body,
        out_shape=jax.ShapeDtypeStruct((n, 2), f32),
    )(part2, y3, b2)


def _sc_logits1(xl, xr, src, dst, att1, e_total, blk):
    ew = e_total // NW
    nchunk = ew // blk

    @functools.partial(
        pl.kernel,
        mesh=_mesh(),
        compiler_params=pltpu.CompilerParams(needs_layout_passes=False, use_tc_tiling_on_sc=False),
        out_type=[jax.ShapeDtypeStruct((e_total * 4,), f32),
                  jax.ShapeDtypeStruct((NW * 16,), f32)],
        scratch_types=[
            pltpu.VMEM((blk,), jnp.int32),
            pltpu.VMEM((blk,), jnp.int32),
            pltpu.VMEM((blk, 64), f32),
            pltpu.VMEM((blk, 64), f32),
            pltpu.VMEM((blk * 4,), f32),
            pltpu.VMEM((4, 16), f32),
            pltpu.VMEM((16,), f32),
            pltpu.SemaphoreType.DMA,
        ],
    )
    def k(xl_h, xr_h, src_h, dst_h, att_h, lg_h, wmax_h,
          srcv, dstv, xlv, xrv, lgv, attv, wmv, sem):
        c = lax.axis_index("c")
        s = lax.axis_index("s")
        wid = c * NS + s
        pltpu.sync_copy(att_h, attv)
        att_rows = [attv[h, :] for h in range(4)]
        base0 = wid * ew
        lane = lax.iota(jnp.int32, 16)
        diag = [(lane + c0) & 15 for c0 in range(16)]

        def chunk(i, vmax):
            base = base0 + i * blk
            pltpu.sync_copy(src_h.at[pl.ds(base, blk)], srcv)
            pltpu.sync_copy(dst_h.at[pl.ds(base, blk)], dstv)
            cp1 = pltpu.async_copy(xl_h.at[srcv], xlv, sem)
            cp2 = pltpu.async_copy(xr_h.at[dstv], xrv, sem)
            cp1.wait()
            cp2.wait()

            def grp(g, vm):
                eidx = lane + g * 16
                sidx = eidx * 4
                for h in range(4):
                    acc = jnp.zeros((16,), f32)
                    for c0 in range(16):
                        cidx = diag[c0] + h * 16
                        attr = _shuf_idx(att_rows[h], diag[c0])
                        z = (plsc.load_gather(xlv, [eidx, cidx])
                             + plsc.load_gather(xrv, [eidx, cidx]))
                        z = jnp.maximum(z, 0.2 * z)
                        acc = acc + z * attr
                    plsc.store_scatter(lgv, [sidx + h], acc)
                    vm = jnp.maximum(vm, acc)
                return vm

            vmax = lax.fori_loop(0, blk // 16, grp, vmax)
            pltpu.sync_copy(lgv, lg_h.at[pl.ds(base * 4, blk * 4)])
            return vmax

        vmax = lax.fori_loop(0, nchunk, chunk,
                             jnp.full((16,), -3e38, dtype=f32))
        wmv[...] = vmax
        pltpu.sync_copy(wmv, wmax_h.at[pl.ds(wid * 16, 16)])

    return k(xl, xr, src, dst, att1)


def _sc_accum1(xl, src, dst, lg, wmax, zin, e_total, n, blk):
    ew = e_total // NW
    nchunk = ew // blk
    stripe = n // NS

    @functools.partial(
        pl.kernel,
        mesh=_mesh(),
        compiler_params=pltpu.CompilerParams(needs_layout_passes=False, use_tc_tiling_on_sc=False),
        out_type=[jax.ShapeDtypeStruct((NC, n, 80), f32)],
        scratch_types=[
            pltpu.VMEM((blk,), jnp.int32),
            pltpu.VMEM((blk,), jnp.int32),
            pltpu.VMEM((blk, 64), f32),
            pltpu.VMEM((blk * 4,), f32),
            pltpu.VMEM((blk * 4 + 16,), f32),
            pltpu.VMEM((blk, 80), f32),
            pltpu.VMEM((NW * 16,), f32),
            pltpu.VMEM_SHARED((n, 80), f32),
            pltpu.SemaphoreType.DMA,
        ],
    )
    def k(xl_h, src_h, dst_h, lg_h, wmax_h, zin_h, out_h,
          srcv, dstv, xlv, lgcv, exf, accv, wmv, sh, sem):
        c = lax.axis_index("c")
        s = lax.axis_index("s")
        wid = c * NS + s
        lane = lax.iota(jnp.int32, 16)

        pltpu.sync_copy(wmax_h, wmv)

        def mx(j, vm):
            return jnp.maximum(vm, wmv[pl.ds(j * 16, 16)])

        vm = lax.fori_loop(0, NW, mx, jnp.full((16,), -3e38, dtype=f32))
        gmaxv = _bcast_max(vm, lane)

        pltpu.sync_copy(zin_h.at[pl.ds(s * stripe, stripe)],
                        sh.at[pl.ds(s * stripe, stripe)])
        pltpu.sync_copy(zin_h.at[pl.ds(0, blk)], accv)
        plsc.subcore_barrier()

        base0 = wid * ew

        def chunk(i, _):
            base = base0 + i * blk
            pltpu.sync_copy(src_h.at[pl.ds(base, blk)], srcv)
            pltpu.sync_copy(dst_h.at[pl.ds(base, blk)], dstv)
            pltpu.sync_copy(lg_h.at[pl.ds(base * 4, blk * 4)], lgcv)
            cp = pltpu.async_copy(xl_h.at[srcv], xlv, sem)

            def expj(j, _2):
                exf[pl.ds(j * 16, 16)] = jnp.exp(
                    lgcv[pl.ds(j * 16, 16)] - gmaxv)
                return 0

            lax.fori_loop(0, blk * 4 // 16, expj, 0)
            cp.wait()

            def edge(e, _2):
                exvec = exf[pl.ds(e * 4, 16)]
                for h in range(4):
                    accv[e, pl.ds(h * 16, 16)] = (
                        xlv[e, pl.ds(h * 16, 16)] * exvec[h])
                accv[e, pl.ds(64, 16)] = exvec
                return 0

            lax.fori_loop(0, blk, edge, 0)
            pltpu.sync_copy(accv, sh.at[dstv], add=True)
            return 0

        lax.fori_loop(0, nchunk, chunk, 0)
        plsc.subcore_barrier()
        pltpu.sync_copy(sh.at[pl.ds(s * stripe, stripe)],
                        out_h.at[c, pl.ds(s * stripe, stripe)])

    return k(xl, src, dst, lg, wmax, zin)[0]


def _sc_logits2(t2, src, dst, att2, e_total, blk):
    ew = e_total // NW
    nchunk = ew // blk
    ngrp = blk // 16

    @functools.partial(
        pl.kernel,
        mesh=_mesh(),
        compiler_params=pltpu.CompilerParams(needs_layout_passes=False, use_tc_tiling_on_sc=False),
        out_type=[jax.ShapeDtypeStruct((e_total,), f32),
                  jax.ShapeDtypeStruct((NW * 16,), f32)],
        scratch_types=[
            pltpu.VMEM((blk,), jnp.int32),
            pltpu.VMEM((blk,), jnp.int32),
            pltpu.VMEM((blk, 16), f32),
            pltpu.VMEM((blk, 16), f32),
            pltpu.VMEM((blk,), f32),
            pltpu.VMEM((16,), f32),
            pltpu.VMEM((16,), f32),
            pltpu.SemaphoreType.DMA,
        ],
    )
    def k(t2_h, src_h, dst_h, att_h, lg_h, wmax_h,
          srcv, dstv, tsv, tdv, lgv, attv, wmv, sem):
        c = lax.axis_index("c")
        s = lax.axis_index("s")
        wid = c * NS + s
        pltpu.sync_copy(att_h, attv)
        av = attv[...]
        t0 = av[0]
        t1 = av[1]
        iota = lax.iota(jnp.int32, 16)
        c0 = jnp.zeros((16,), jnp.int32)
        c1 = c0 + 1
        c2 = c0 + 2
        c3 = c0 + 3
        base0 = wid * ew

        def chunk(i, vmax):
            base = base0 + i * blk
            pltpu.sync_copy(src_h.at[pl.ds(base, blk)], srcv)
            pltpu.sync_copy(dst_h.at[pl.ds(base, blk)], dstv)
            cp1 = pltpu.async_copy(t2_h.at[srcv], tsv, sem)
            cp2 = pltpu.async_copy(t2_h.at[dstv], tdv, sem)
            cp1.wait()
            cp2.wait()

            def grp(g, vm):
                i0 = iota + g * 16
                a0 = plsc.load_gather(tsv, [i0, c0])
                a1 = plsc.load_gather(tsv, [i0, c1])
                d0 = plsc.load_gather(tdv, [i0, c2])
                d1 = plsc.load_gather(tdv, [i0, c3])
                z0 = a0 + d0
                z0 = jnp.maximum(z0, 0.2 * z0)
                z1 = a1 + d1
                z1 = jnp.maximum(z1, 0.2 * z1)
                lgvec = z0 * t0 + z1 * t1
                lgv[pl.ds(g * 16, 16)] = lgvec
                return jnp.maximum(vm, lgvec)

            vmax = lax.fori_loop(0, ngrp, grp, vmax)
            pltpu.sync_copy(lgv, lg_h.at[pl.ds(base, blk)])
            return vmax

        vmax = lax.fori_loop(0, nchunk, chunk,
                             jnp.full((16,), -3e38, dtype=f32))
        wmv[...] = vmax
        pltpu.sync_copy(wmv, wmax_h.at[pl.ds(wid * 16, 16)])

    return k(t2, src, dst, att2)


def _sc_accum2(t2, src, dst, lg, wmax, zin, e_total, n, blk):
    ew = e_total // NW
    nchunk = ew // blk
    ngrp = blk // 16
    stripe = n // NS

    @functools.partial(
        pl.kernel,
        mesh=_mesh(),
        compiler_params=pltpu.CompilerParams(needs_layout_passes=False, use_tc_tiling_on_sc=False),
        out_type=[jax.ShapeDtypeStruct((NC, n, 8), f32)],
        scratch_types=[
            pltpu.VMEM((blk,), jnp.int32),
            pltpu.VMEM((blk,), jnp.int32),
            pltpu.VMEM((blk, 16), f32),
            pltpu.VMEM((blk,), f32),
            pltpu.VMEM((blk, 8), f32),
            pltpu.VMEM((NW * 16,), f32),
            pltpu.VMEM_SHARED((n, 8), f32),
            pltpu.SemaphoreType.DMA,
        ],
    )
    def k(t2_h, src_h, dst_h, lg_h, wmax_h, zin_h, out_h,
          srcv, dstv, tsv, lgv, accv, wmv, sh, sem):
        c = lax.axis_index("c")
        s = lax.axis_index("s")
        wid = c * NS + s

        pltpu.sync_copy(wmax_h, wmv)

        def mx(j, vm):
            return jnp.maximum(vm, wmv[pl.ds(j * 16, 16)])

        vm = lax.fori_loop(0, NW, mx, jnp.full((16,), -3e38, dtype=f32))
        gmaxv = _bcast_max(vm, lax.iota(jnp.int32, 16))

        pltpu.sync_copy(zin_h.at[pl.ds(s * stripe, stripe)],
                        sh.at[pl.ds(s * stripe, stripe)])
        pltpu.sync_copy(zin_h.at[pl.ds(0, blk)], accv)
        plsc.subcore_barrier()

        iota = lax.iota(jnp.int32, 16)
        c0 = jnp.zeros((16,), jnp.int32)
        c1 = c0 + 1
        c2 = c0 + 2
        c3 = c0 + 3
        base0 = wid * ew

        def chunk(i, _):
            base = base0 + i * blk
            pltpu.sync_copy(src_h.at[pl.ds(base, blk)], srcv)
            pltpu.sync_copy(dst_h.at[pl.ds(base, blk)], dstv)
            pltpu.sync_copy(lg_h.at[pl.ds(base, blk)], lgv)
            cp = pltpu.async_copy(t2_h.at[srcv], tsv, sem)
            cp.wait()

            def grp(g, _2):
                i0 = iota + g * 16
                ex = jnp.exp(lgv[pl.ds(g * 16, 16)] - gmaxv)
                a0 = plsc.load_gather(tsv, [i0, c0])
                a1 = plsc.load_gather(tsv, [i0, c1])
                plsc.store_scatter(accv, [i0, c0], a0 * ex)
                plsc.store_scatter(accv, [i0, c1], a1 * ex)
                plsc.store_scatter(accv, [i0, c2], ex)
                plsc.store_scatter(accv, [i0, c3], ex)
                return 0

            lax.fori_loop(0, ngrp, grp, 0)
            pltpu.sync_copy(accv, sh.at[dstv], add=True)
            return 0

        lax.fori_loop(0, nchunk, chunk, 0)
        plsc.subcore_barrier()
        pltpu.sync_copy(sh.at[pl.ds(s * stripe, stripe)],
                        out_h.at[c, pl.ds(s * stripe, stripe)])

    return k(t2, src, dst, lg, wmax, zin)[0]


USE_SC1 = True
USE_SA1 = True
USE_SC2 = True
USE_SA2 = True
USE_TC = True


def _jnp_logits1(xl, xr, src, dst, att1, e_total):
    xls = xl[src].reshape(e_total, 4, 16)
    xrd = xr[dst].reshape(e_total, 4, 16)
    z = xls + xrd
    z = jnp.maximum(z, 0.2 * z)
    lg = (z * att1[None]).sum(-1)
    wmax = jnp.full((NW * 16,), lg.max(), f32)
    return lg.T.reshape(-1), wmax


def _jnp_accum1(xl, src, dst, lgp, wmax, e_total, n):
    lg = lgp.reshape(4, e_total).T
    ex = jnp.exp(lg - wmax.max())
    xls = xl[src].reshape(e_total, 4, 16)
    num = jax.ops.segment_sum((xls * ex[:, :, None]).reshape(e_total, 64),
                              dst, num_segments=n)
    den = jax.ops.segment_sum(ex, dst, num_segments=n)
    part = jnp.zeros((NC, n, 80), f32)
    part = part.at[0, :, :64].set(num).at[0, :, 64:68].set(den)
    return part


def _jnp_mid(part1, b1, gamma, beta, w24, w3, b3, n):
    p = part1[0] + part1[1]
    num = p[:, :64]
    den = p[:, 64:68]
    denb = jnp.repeat(1.0 / (den + 1e-16), 16, axis=1)
    pre = num * denb + b1
    h = jnp.where(pre > 0, pre, jnp.exp(pre) - 1)
    mean = h.mean(0); cent = h - mean; var = (cent * cent).mean(0)
    hbn = gamma * cent / jnp.sqrt(var + 1e-5) + beta
    return hbn @ w24, hbn @ w3 + b3


def _jnp_logits2(t2, src, dst, att2p, e_total):
    ts = t2[src]; td = t2[dst]
    z0 = ts[:, 0] + td[:, 2]; z0 = jnp.maximum(z0, 0.2 * z0)
    z1 = ts[:, 1] + td[:, 3]; z1 = jnp.maximum(z1, 0.2 * z1)
    lg2 = z0 * att2p[0] + z1 * att2p[1]
    return lg2, jnp.full((NW * 16,), lg2.max(), f32)


def _jnp_accum2(t2, src, dst, lg2, wmax, e_total, n):
    ex2 = jnp.exp(lg2 - wmax.max())
    ts = t2[src]
    num2 = jax.ops.segment_sum(ts[:, :2] * ex2[:, None], dst, num_segments=n)
    den2 = jax.ops.segment_sum(ex2, dst, num_segments=n)
    part = jnp.zeros((NC, n, 8), f32)
    part = part.at[0, :, :2].set(num2)
    part = part.at[0, :, 2].set(den2).at[0, :, 3].set(den2)
    return part


def _jnp_final(part2, y3, b2):
    p = part2[0] + part2[1]
    zf = p[:, 0:2] / (p[:, 2:4] + 1e-16) + b2 + y3
    return jnp.where(zf > 0, zf, jnp.exp(zf) - 1)


def kernel(x, edge_index, W1l, W1r, att1, b1, gamma, beta, W2l, W2r, att2,
           b2, W3, b3):
    n, d = x.shape
    e_total = edge_index.shape[1]
    blk = 400
    assert e_total % (NW * blk) == 0 and n % NS == 0

    src = edge_index[0]
    dst = edge_index[1]
    w1cat = jnp.concatenate([W1l, W1r], axis=1)
    w24 = jnp.concatenate(
        [W2l, W2r, jnp.zeros((64, 12), f32)], axis=1)
    expander = jnp.repeat(jnp.eye(4, dtype=f32), 16, axis=1)
    zin1 = jnp.zeros((n, 80), f32)
    zin2 = jnp.zeros((n, 8), f32)

    att2p = jnp.zeros((16,), f32).at[:2].set(att2.reshape(-1))
    if USE_TC:
        xl, xr = _tc_project(x, w1cat, n, d)
    else:
        y = x @ w1cat
        xl, xr = y[:, :64], y[:, 64:]
    if USE_SC1:
        lg1, wmax1 = _sc_logits1(xl, xr, src, dst, att1, e_total, blk)
    else:
        lg1, wmax1 = _jnp_logits1(xl, xr, src, dst, att1, e_total)
    if USE_SA1:
        part1 = _sc_accum1(xl, src, dst, lg1, wmax1, zin1, e_total, n, blk)
    else:
        part1 = _jnp_accum1(xl, src, dst, lg1, wmax1, e_total, n)
    if USE_TC:
        t2, y3 = _tc_mid(part1, b1, gamma, beta, expander, w24, W3, b3, n)
    else:
        t2, y3 = _jnp_mid(part1, b1, gamma, beta, w24, W3, b3, n)
    if USE_SC2:
        lg2, wmax2 = _sc_logits2(t2, src, dst, att2p, e_total, blk)
    else:
        lg2, wmax2 = _jnp_logits2(t2, src, dst, att2p, e_total)
    if USE_SA2:
        part2 = _sc_accum2(t2, src, dst, lg2, wmax2, zin2, e_total, n, blk)
    else:
        part2 = _jnp_accum2(t2, src, dst, lg2, wmax2, e_total, n)
    if USE_TC:
        return _tc_final(part2, y3, b2, n)
    return _jnp_final(part2, y3, b2)
